# Initial kernel scaffold; baseline (speedup 1.0000x reference)
import jax
import jax.numpy as jnp
from jax.experimental import pallas as pl


def kernel(x, edge_index, W, b):
    raise NotImplementedError("write your pallas kernel here")


# speedup vs baseline: 11.2380x; 11.2380x over previous
import jax
import jax.numpy as jnp
from jax import lax
from jax.experimental import pallas as pl
from jax.experimental.pallas import tpu as pltpu, tpu_sc as plsc

N = 10000
E = 320000
D = 128
NPAD = 10240
NC = 2
NS = 16
NW = NC * NS
EPW = E // NW
CB = 40
NCHUNK = EPW // CB
NBLK = 5
BCH = NCHUNK // NBLK
RPT = NPAD // NS


def _sc_body(x_hbm, src_hbm, dst_hbm, acc_out, cnt_out,
             idx_s, idx_d, s0, s1, d0, d1, czero, ones_v,
             sem_s0, sem_s1, sem_d0, sem_d1, acc_sh, cnt_sh):
    c = lax.axis_index("c")
    s = lax.axis_index("s")
    wid = c * NS + s

    def _zrow(r, _):
        for j in range(D // 16):
            s0[r, pl.ds(j * 16, 16)] = jnp.zeros((16,), jnp.float32)
        return 0
    lax.fori_loop(0, CB, _zrow, 0)

    def _zc(r, _):
        czero[pl.ds(r * 16, 16)] = jnp.zeros((16,), jnp.float32)
        return 0
    lax.fori_loop(0, RPT // 16, _zc, 0)

    for j in range(3):
        ones_v[pl.ds(j * 16, 16)] = jnp.ones((16,), jnp.float32)

    base = s * RPT
    for t in range(RPT // CB):
        pltpu.sync_copy(s0, acc_sh.at[pl.ds(base + t * CB, CB)])
    pltpu.sync_copy(czero, cnt_sh.at[pl.ds(base, RPT)])
    plsc.subcore_barrier()

    def _wait(buf, sem):
        pltpu.make_async_copy(x_hbm.at[pl.ds(0, CB)], buf, sem).wait()

    def _compute(sb, db):
        def _row(r, _):
            for j in range(D // 16):
                sl = pl.ds(j * 16, 16)
                sb[r, sl] = jnp.abs(sb[r, sl] - db[r, sl])
            return 0
        lax.fori_loop(0, CB, _row, 0)

    def _block(blk, _):
        pltpu.sync_copy(src_hbm.at[wid, blk], idx_s)
        pltpu.sync_copy(dst_hbm.at[wid, blk], idx_d)
        pltpu.async_copy(x_hbm.at[idx_s.at[0]], s0, sem_s0)
        pltpu.async_copy(x_hbm.at[idx_d.at[0]], d0, sem_d0)

        def _pair(k2, __):
            e = 2 * k2
            _wait(s0, sem_s0)
            _wait(d0, sem_d0)
            pltpu.async_copy(x_hbm.at[idx_s.at[e + 1]], s1, sem_s1)
            pltpu.async_copy(x_hbm.at[idx_d.at[e + 1]], d1, sem_d1)
            _compute(s0, d0)
            pltpu.sync_copy(s0, acc_sh.at[idx_s.at[e]], add=True)
            pltpu.sync_copy(ones_v.at[pl.ds(0, CB)],
                            cnt_sh.at[idx_s.at[e]], add=True)
            _wait(s1, sem_s1)
            _wait(d1, sem_d1)

            @pl.when(k2 < BCH // 2 - 1)
            def _():
                pltpu.async_copy(x_hbm.at[idx_s.at[e + 2]], s0, sem_s0)
                pltpu.async_copy(x_hbm.at[idx_d.at[e + 2]], d0, sem_d0)

            _compute(s1, d1)
            pltpu.sync_copy(s1, acc_sh.at[idx_s.at[e + 1]], add=True)
            pltpu.sync_copy(ones_v.at[pl.ds(0, CB)],
                            cnt_sh.at[idx_s.at[e + 1]], add=True)
            return 0
        lax.fori_loop(0, BCH // 2, _pair, 0)
        return 0
    lax.fori_loop(0, NBLK, _block, 0)

    plsc.subcore_barrier()
    pltpu.sync_copy(acc_sh.at[pl.ds(base, RPT)],
                    acc_out.at[c].at[pl.ds(base, RPT)])
    pltpu.sync_copy(cnt_sh.at[pl.ds(base, RPT)],
                    cnt_out.at[pl.ds(c * NPAD + base, RPT)])


_sc_gather_scatter = pl.kernel(
    _sc_body,
    out_type=(
        jax.ShapeDtypeStruct((NC, NPAD, D), jnp.float32),
        jax.ShapeDtypeStruct((NC * NPAD,), jnp.float32),
    ),
    mesh=plsc.VectorSubcoreMesh(core_axis_name="c", subcore_axis_name="s"),
    scratch_types=[
        pltpu.VMEM((BCH, CB), jnp.int32),
        pltpu.VMEM((BCH, CB), jnp.int32),
        pltpu.VMEM((CB, D), jnp.float32),
        pltpu.VMEM((CB, D), jnp.float32),
        pltpu.VMEM((CB, D), jnp.float32),
        pltpu.VMEM((CB, D), jnp.float32),
        pltpu.VMEM((RPT,), jnp.float32),
        pltpu.VMEM((48,), jnp.float32),
        pltpu.SemaphoreType.DMA,
        pltpu.SemaphoreType.DMA,
        pltpu.SemaphoreType.DMA,
        pltpu.SemaphoreType.DMA,
        pltpu.VMEM_SHARED((NPAD, D), jnp.float32),
        pltpu.VMEM_SHARED((NPAD,), jnp.float32),
    ],
)


BLK = 512


def _tc_body(x_ref, acc_ref, cnt_ref, w_ref, b_ref, o_ref):
    a = x_ref[...] + acc_ref[0] + acc_ref[1]
    denom = cnt_ref[...].sum(axis=1, keepdims=True) + 1.0
    m = a / denom
    o_ref[...] = lax.dot_general(
        m, w_ref[...], (((1,), (1,)), ((), ())),
        preferred_element_type=jnp.float32) + b_ref[...]


_tc_combine = pl.pallas_call(
    _tc_body,
    grid=(NPAD // BLK,),
    in_specs=[
        pl.BlockSpec((BLK, D), lambda i: (i, 0)),
        pl.BlockSpec((NC, BLK, D), lambda i: (0, i, 0)),
        pl.BlockSpec((BLK, NC), lambda i: (i, 0)),
        pl.BlockSpec((D, D), lambda i: (0, 0)),
        pl.BlockSpec((1, D), lambda i: (0, 0)),
    ],
    out_specs=pl.BlockSpec((BLK, D), lambda i: (i, 0)),
    out_shape=jax.ShapeDtypeStruct((NPAD, D), jnp.float32),
    compiler_params=pltpu.CompilerParams(
        dimension_semantics=("arbitrary",)),
)


def kernel(x, edge_index, W, b):
    src = edge_index[0].reshape(NW, NBLK, BCH, CB)
    dst = edge_index[1].reshape(NW, NBLK, BCH, CB)
    acc, cnt = _sc_gather_scatter(x, src, dst)
    xp = jnp.pad(x, ((0, NPAD - N), (0, 0)))
    out = _tc_combine(xp, acc, cnt.reshape(NC, NPAD).T, W, b.reshape(1, D))
    return out[:N]
